# baseline (speedup 1.0000x reference)
import functools
import math

import jax
import jax.numpy as jnp
from jax import lax
from jax.experimental import pallas as pl
from jax.experimental.pallas import tpu as pltpu
from jax.experimental.pallas import tpu_sc as plsc

S = 64.0
MARGIN = 0.5
COS_M = math.cos(MARGIN)
SIN_M = math.sin(MARGIN)
THETA = math.cos(math.pi - MARGIN)
SINMM = math.sin(math.pi - MARGIN) * MARGIN

B = 1024
N = 100000

_NC = 2
_NS = 16
_NW = _NC * _NS
_RPW = B // _NW
_L = 16


def _sc_margin_body(flat_hbm, labels_hbm, out_hbm, lab_v, idx_v, val_v, fin_v, sem):
    wid = lax.axis_index("s") * _NC + lax.axis_index("c")
    base = wid * _RPW
    pltpu.sync_copy(labels_hbm.at[pl.ds(base, _RPW)], lab_v)
    for c in range(_RPW // _L):
        lab = lab_v[pl.ds(c * _L, _L)]
        safe = jnp.maximum(lab, 0)
        rows = base + c * _L + lax.broadcasted_iota(jnp.int32, (_L,), 0)
        idx_v[pl.ds(c * _L, _L)] = rows * N + safe
    pltpu.async_copy(flat_hbm.at[idx_v], val_v, sem).wait()
    for c in range(_RPW // _L):
        t = val_v[pl.ds(c * _L, _L)]
        u = 1.0 - t * t
        i = lax.bitcast_convert_type(u, jnp.int32)
        i = 0x5F3759DF - lax.shift_right_logical(i, 1)
        y = lax.bitcast_convert_type(i, jnp.float32)
        for _ in range(3):
            y = y * (1.5 - 0.5 * u * y * y)
        sin_t = u * y
        cosm = t * COS_M - sin_t * SIN_M
        fin = jnp.where(t > THETA, cosm, t - SINMM)
        fin_v[pl.ds(c * _L, _L)] = fin
    pltpu.sync_copy(fin_v, out_hbm.at[pl.ds(base, _RPW)])


@functools.cache
def _sc_margin():
    return pl.kernel(
        _sc_margin_body,
        mesh=plsc.VectorSubcoreMesh(core_axis_name="c", subcore_axis_name="s"),
        out_type=jax.ShapeDtypeStruct((B,), jnp.float32),
        scratch_types=[
            pltpu.VMEM((_RPW,), jnp.int32),
            pltpu.VMEM((_RPW,), jnp.int32),
            pltpu.VMEM((_RPW,), jnp.float32),
            pltpu.VMEM((_RPW,), jnp.float32),
            pltpu.SemaphoreType.DMA,
        ],
    )


_RB = 16


def _tc_body(lab_ref, x_ref, o_ref):
    x = x_ref[...]
    col = lax.broadcasted_iota(jnp.int32, x.shape, 1)
    mask = col == lab_ref[...]
    tgt = jnp.sum(jnp.where(mask, x, 0.0), axis=1, keepdims=True)
    sin_t = jnp.sqrt(1.0 - tgt * tgt)
    cosm = tgt * COS_M - sin_t * SIN_M
    fin = jnp.where(tgt > THETA, cosm, tgt - SINMM)
    o_ref[...] = jnp.where(mask, fin, x) * S


def _tc_scale_merge(logits, labels2d):
    grid = (B // _RB,)
    return pl.pallas_call(
        _tc_body,
        grid=grid,
        in_specs=[
            pl.BlockSpec((_RB, 1), lambda i: (i, 0)),
            pl.BlockSpec((_RB, N), lambda i: (i, 0)),
        ],
        out_specs=pl.BlockSpec((_RB, N), lambda i: (i, 0)),
        out_shape=jax.ShapeDtypeStruct((B, N), jnp.float32),
    )(labels2d, logits)


_TRW = (B // 8) // _NW
_CW = 512
_NCH = 98304 // _CW
_TAIL0 = _NCH * _CW
_TW = N - _TAIL0
_NBUF = 8


def _margin16(tsc):
    t = tsc * (1.0 / S)
    u = 1.0 - t * t
    i = lax.bitcast_convert_type(u, jnp.int32)
    i = 0x5F3759DF - lax.shift_right_logical(i, 1)
    y = lax.bitcast_convert_type(i, jnp.float32)
    for _ in range(3):
        y = y * (1.5 - 0.5 * u * y * y)
    sin_t = u * y
    cosm = t * COS_M - sin_t * SIN_M
    fin = jnp.where(t > THETA, cosm, t - SINMM)
    return fin * S


def _sc_body(x_hbm, lab_hbm, o_hbm, lab_v, *rest):
    ibufs = tuple(zip(rest[:_NBUF], rest[2 * _NBUF + 1:3 * _NBUF + 1]))
    obufs = tuple(zip(rest[_NBUF:2 * _NBUF], rest[3 * _NBUF + 1:4 * _NBUF + 1]))
    tbuf = rest[2 * _NBUF]
    tsem = rest[4 * _NBUF + 1]
    wid = lax.axis_index("s") * _NC + lax.axis_index("c")
    row0 = wid * (_TRW * 8)
    pltpu.sync_copy(lab_hbm.at[pl.ds(row0, _TRW * 8)], lab_v)
    lane = lax.broadcasted_iota(jnp.int32, (16,), 0)
    rowin = lane % 8
    Q = _TRW * _NCH

    def addr(q):
        t = q // _NCH
        return row0 + t * 8, (q - t * _NCH) * _CW, t

    def fix(q, ob, c, w, t):
        lab8 = plsc.load_gather(lab_v, [t * 8 + rowin])
        inwin = (lab8 >= c) & (lab8 < c + w) & (lane < 8)
        colin = jnp.clip(lab8 - c, 0, w - 1)
        tsc = plsc.load_gather(ob, [rowin, colin], mask=inwin)
        plsc.store_scatter(ob, [rowin, colin], _margin16(tsc), mask=inwin)

    def start_in(q, ib, sem):
        r, c, _ = addr(q)
        pltpu.make_async_copy(
            x_hbm.at[pl.ds(r, 8), pl.ds(c, _CW)], ib, sem).start()

    def wait_in(ib, sem):
        pltpu.make_async_copy(
            x_hbm.at[pl.ds(0, 8), pl.ds(0, _CW)], ib, sem).wait()

    def compute(q, ib, ob):
        _, c, t = addr(q)
        for j in range(_CW // 16):
            for i in range(8):
                ob[i, pl.ds(j * 16, 16)] = ib[i, pl.ds(j * 16, 16)] * S
        fix(q, ob, c, _CW, t)

    def start_out(q, ob, sem):
        r, c, _ = addr(q)
        pltpu.make_async_copy(
            ob, o_hbm.at[pl.ds(r, 8), pl.ds(c, _CW)], sem).start()

    def wait_out(ob, sem):
        pltpu.make_async_copy(
            ob, o_hbm.at[pl.ds(0, 8), pl.ds(0, _CW)], sem).wait()

    for p in range(_NBUF - 1):
        start_in(p, *ibufs[p])

    def loop_body(m, _):
        for b in range(_NBUF):
            q = _NBUF * m + b
            ib, isem = ibufs[b]
            ob, osem = obufs[b]
            nib, nisem = ibufs[(b + _NBUF - 1) % _NBUF]
            pl.when(q + _NBUF - 1 < Q)(
                lambda: start_in(q + _NBUF - 1, nib, nisem))
            wait_in(ib, isem)
            pl.when(q >= _NBUF)(lambda: wait_out(ob, osem))
            compute(q, ib, ob)
            start_out(q, ob, osem)
        return 0

    lax.fori_loop(0, Q // _NBUF, loop_body, 0)
    for p in range(_NBUF):
        wait_out(*obufs[p])

    for t in range(_TRW):
        r = row0 + t * 8
        pltpu.async_copy(
            x_hbm.at[pl.ds(r, 8), pl.ds(_TAIL0, _TW)], tbuf, tsem).wait()

        def tbody(j, _):
            for i in range(8):
                tbuf[i, pl.ds(j * 16, 16)] = tbuf[i, pl.ds(j * 16, 16)] * S
            return 0

        lax.fori_loop(0, _TW // 16, tbody, 0, unroll=4)
        fix(0, tbuf, _TAIL0, _TW, t)
        pltpu.async_copy(
            tbuf, o_hbm.at[pl.ds(r, 8), pl.ds(_TAIL0, _TW)], tsem).wait()


@functools.cache
def _sc_scale():
    return pl.kernel(
        _sc_body,
        mesh=plsc.VectorSubcoreMesh(core_axis_name="c", subcore_axis_name="s"),
        out_type=jax.ShapeDtypeStruct((B, N), jnp.float32),
        scratch_types=(
            [pltpu.VMEM((_TRW * 8,), jnp.int32)]
            + [pltpu.VMEM((8, _CW), jnp.float32)] * (2 * _NBUF)
            + [pltpu.VMEM((8, _TW), jnp.float32)]
            + [pltpu.SemaphoreType.DMA] * (2 * _NBUF + 1)
        ),
        compiler_params=pltpu.CompilerParams(
            use_tc_tiling_on_sc=True, needs_layout_passes=False
        ),
    )


@jax.jit
def kernel(logits, labels):
    labels = labels.astype(jnp.int32)
    return _sc_scale()(logits, labels)


# speedup vs baseline: 1.1493x; 1.1493x over previous
import functools
import math

import jax
import jax.numpy as jnp
from jax import lax
from jax.experimental import pallas as pl
from jax.experimental.pallas import tpu as pltpu
from jax.experimental.pallas import tpu_sc as plsc

S = 64.0
MARGIN = 0.5
COS_M = math.cos(MARGIN)
SIN_M = math.sin(MARGIN)
THETA = math.cos(math.pi - MARGIN)
SINMM = math.sin(math.pi - MARGIN) * MARGIN

B = 1024
N = 100000

_NC = 2
_NS = 16
_NW = _NC * _NS
_RPW = B // _NW
_L = 16


def _sc_margin_body(flat_hbm, labels_hbm, out_hbm, lab_v, idx_v, val_v, fin_v, sem):
    wid = lax.axis_index("s") * _NC + lax.axis_index("c")
    base = wid * _RPW
    pltpu.sync_copy(labels_hbm.at[pl.ds(base, _RPW)], lab_v)
    for c in range(_RPW // _L):
        lab = lab_v[pl.ds(c * _L, _L)]
        safe = jnp.maximum(lab, 0)
        rows = base + c * _L + lax.broadcasted_iota(jnp.int32, (_L,), 0)
        idx_v[pl.ds(c * _L, _L)] = rows * N + safe
    pltpu.async_copy(flat_hbm.at[idx_v], val_v, sem).wait()
    for c in range(_RPW // _L):
        t = val_v[pl.ds(c * _L, _L)]
        u = 1.0 - t * t
        i = lax.bitcast_convert_type(u, jnp.int32)
        i = 0x5F3759DF - lax.shift_right_logical(i, 1)
        y = lax.bitcast_convert_type(i, jnp.float32)
        for _ in range(3):
            y = y * (1.5 - 0.5 * u * y * y)
        sin_t = u * y
        cosm = t * COS_M - sin_t * SIN_M
        fin = jnp.where(t > THETA, cosm, t - SINMM)
        fin_v[pl.ds(c * _L, _L)] = fin
    pltpu.sync_copy(fin_v, out_hbm.at[pl.ds(base, _RPW)])


@functools.cache
def _sc_margin():
    return pl.kernel(
        _sc_margin_body,
        mesh=plsc.VectorSubcoreMesh(core_axis_name="c", subcore_axis_name="s"),
        out_type=jax.ShapeDtypeStruct((B,), jnp.float32),
        scratch_types=[
            pltpu.VMEM((_RPW,), jnp.int32),
            pltpu.VMEM((_RPW,), jnp.int32),
            pltpu.VMEM((_RPW,), jnp.float32),
            pltpu.VMEM((_RPW,), jnp.float32),
            pltpu.SemaphoreType.DMA,
        ],
    )


_RB = 32


def _tc_body(lab_ref, x_ref, o_ref):
    x = x_ref[...]
    col = lax.broadcasted_iota(jnp.int32, x.shape, 1)
    mask = col == lab_ref[...]
    tgt = jnp.sum(jnp.where(mask, x, 0.0), axis=1, keepdims=True)
    sin_t = jnp.sqrt(1.0 - tgt * tgt)
    cosm = tgt * COS_M - sin_t * SIN_M
    fin = jnp.where(tgt > THETA, cosm, tgt - SINMM)
    o_ref[...] = jnp.where(mask, fin, x) * S


def _tc_scale_merge(logits, labels2d):
    grid = (B // _RB,)
    return pl.pallas_call(
        _tc_body,
        grid=grid,
        in_specs=[
            pl.BlockSpec((_RB, 1), lambda i: (i, 0)),
            pl.BlockSpec((_RB, N), lambda i: (i, 0)),
        ],
        out_specs=pl.BlockSpec((_RB, N), lambda i: (i, 0)),
        out_shape=jax.ShapeDtypeStruct((B, N), jnp.float32),
        compiler_params=pltpu.CompilerParams(vmem_limit_bytes=128 * 1024 * 1024),
    )(labels2d, logits)


@jax.jit
def kernel(logits, labels):
    labels = labels.astype(jnp.int32)
    return _tc_scale_merge(logits, labels.reshape(B, 1))
